# baseline (speedup 1.0000x reference)
import functools

import jax
import jax.numpy as jnp
from jax import lax
from jax.experimental import pallas as pl
from jax.experimental.pallas import tpu as pltpu
from jax.experimental.pallas import tpu_sc as plsc

B = 16384
D0, D1 = 64, 32
D = D0 + D1

NC = 2
NS = 16
NW = NC * NS
BW = B // NW
L = 16
CH = 16
NCH = BW // CH
OUTW = 384
ROWS_PER_W = B * D // OUTW // NW


def _sc_body(item_ids, user_ids, it0, ut0, it1, ut1,
             out_item, out_user,
             idx_i, idx_u, g0, g1, cat, sem):
    wid = lax.axis_index("s") * NC + lax.axis_index("c")
    base = wid * BW
    pltpu.sync_copy(item_ids.at[pl.ds(base, BW)], idx_i)
    pltpu.sync_copy(user_ids.at[pl.ds(base, BW)], idx_u)

    for idx, t0, t1, out in (
        (idx_i, it0, it1, out_item),
        (idx_u, ut0, ut1, out_user),
    ):
        def chunk(c, _, idx=idx, t0=t0, t1=t1):
            def issue(gi, _):
                vec = idx[pl.ds(c * CH + gi * L, L)]
                blk = (vec >> 3) << 3
                for j in range(L):
                    b = pl.multiple_of(blk[j], 8)
                    i = gi * L + j
                    pltpu.async_copy(t0.at[pl.ds(b, 8)],
                                     g0.at[pl.ds(i * 8, 8)], sem)
                    pltpu.async_copy(t1.at[pl.ds(b, 8)],
                                     g1.at[pl.ds(i * 8, 8)], sem)
                return ()

            lax.fori_loop(0, CH // L, issue, ())
            pltpu.make_async_copy(t0.at[pl.ds(0, CH * 8)], g0, sem).wait()
            pltpu.make_async_copy(t1.at[pl.ds(0, CH * 8)], g1, sem).wait()

            def agroup(gi, _):
                svec = idx[pl.ds(c * CH + gi * L, L)] & 7
                for j in range(L):
                    s = svec[j]
                    r0 = (gi * L + j) * 8 + s
                    row = (c * CH + gi * L + j) // 4
                    colbase = D * (j % 4)
                    for k in range(D0 // L):
                        cat[row, pl.ds(colbase + k * L, L)] = \
                            g0[r0, pl.ds(k * L, L)]
                    for k in range(D1 // L):
                        cat[row, pl.ds(colbase + D0 + k * L, L)] = \
                            g1[r0, pl.ds(k * L, L)]
                return ()

            lax.fori_loop(0, CH // L, agroup, ())
            return ()

        lax.fori_loop(0, NCH, chunk, ())
        pltpu.sync_copy(cat, out.at[pl.ds(wid * ROWS_PER_W, ROWS_PER_W)])


def kernel(item_ids, user_ids, item_table_0, user_table_0, item_table_1, user_table_1):
    mesh = plsc.VectorSubcoreMesh(core_axis_name="c", subcore_axis_name="s")
    run = functools.partial(
        pl.kernel,
        out_type=(
            jax.ShapeDtypeStruct((B * D // OUTW, OUTW), jnp.float32),
            jax.ShapeDtypeStruct((B * D // OUTW, OUTW), jnp.float32),
        ),
        mesh=mesh,
        scratch_types=[
            pltpu.VMEM((BW,), jnp.int32),
            pltpu.VMEM((BW,), jnp.int32),
            pltpu.VMEM((CH * 8, D0), jnp.float32),
            pltpu.VMEM((CH * 8, D1), jnp.float32),
            pltpu.VMEM((ROWS_PER_W, OUTW), jnp.float32),
            pltpu.SemaphoreType.DMA,
        ],
    )(_sc_body)
    o_i, o_u = run(item_ids, user_ids, item_table_0, user_table_0,
                   item_table_1, user_table_1)
    return o_i.reshape(B, D), o_u.reshape(B, D)
